# baseline (device time: 18814 ns/iter reference)
import jax
import jax.numpy as jnp
from jax import lax
from jax.experimental import pallas as pl
from jax.experimental.pallas import tpu as pltpu

N_DEV = 4
B, SQ, SKV, HQ, DH = 2, 256, 256, 16, 64
D_MODEL = 512
DBLK = D_MODEL // N_DEV
H_LOC = HQ // N_DEV
NB = SQ // 64


def kernel(x, Wq, K_ext, V_ext, Wo):
    my = lax.axis_index("i")
    xb = x.astype(jnp.bfloat16)
    k_flat = K_ext.reshape(B, SKV, HQ * DH)
    v_flat = V_ext.reshape(B, SKV, HQ * DH)

    def body(x_ref, wq_ref, k_ref, v_ref, wo_ref, out_ref,
             partial_ref, rs_comm, ag_send,
             rs_send_sems, rs_recv_sems, ag_send_sems, ag_recv_sems):
        my_pos = lax.axis_index("i")

        barrier = pltpu.get_barrier_semaphore()
        for d in range(1, N_DEV):
            pl.semaphore_signal(
                barrier, inc=1,
                device_id=((my_pos + d) % N_DEV,),
                device_id_type=pl.DeviceIdType.MESH,
            )

        wq = wq_ref[...].astype(jnp.bfloat16)
        wo = wo_ref[...].astype(jnp.bfloat16)

        rs_rdmas = []
        for b in range(B):
            q = jnp.dot(x_ref[b], wq,
                        preferred_element_type=jnp.float32)
            q4 = (q * 0.125).reshape(SQ, H_LOC, DH).astype(jnp.bfloat16)
            ctxs = []
            for h in range(H_LOC):
                q_blk = q4[:, h, :].reshape(NB, 64, DH)
                k_blk = k_ref[b, :, h * DH:(h + 1) * DH].astype(
                    jnp.bfloat16).reshape(NB, 64, DH)
                s = lax.dot_general(
                    q_blk, k_blk, (((2,), (2,)), ((0,), (0,))),
                    preferred_element_type=jnp.float32,
                )
                w = jnp.exp(s)
                w = w / jnp.sum(w, axis=2, keepdims=True)
                v_blk = v_ref[b, :, h * DH:(h + 1) * DH].astype(
                    jnp.bfloat16).reshape(NB, 64, DH)
                ctx = lax.dot_general(
                    w.astype(jnp.bfloat16), v_blk, (((2,), (1,)), ((0,), (0,))),
                    preferred_element_type=jnp.float32,
                )
                ctxs.append(ctx.reshape(SQ, DH).astype(jnp.bfloat16))
            ctx_b = jnp.concatenate(ctxs, axis=1)

            for t in range(N_DEV):
                p_t = jnp.dot(ctx_b, wo[:, t * DBLK:(t + 1) * DBLK],
                              preferred_element_type=jnp.float32)
                partial_ref[b, t] = p_t.astype(jnp.bfloat16)

            if b == 0:
                pl.semaphore_wait(barrier, N_DEV - 1)

            for d in range(1, N_DEV):
                rdma = pltpu.make_async_remote_copy(
                    src_ref=partial_ref.at[b, (my_pos + d) % N_DEV],
                    dst_ref=rs_comm.at[d - 1, b],
                    send_sem=rs_send_sems.at[d - 1, b],
                    recv_sem=rs_recv_sems.at[d - 1, b],
                    device_id=((my_pos + d) % N_DEV,),
                    device_id_type=pl.DeviceIdType.MESH,
                )
                rdma.start()
                rs_rdmas.append(rdma)

        ag_waits = []
        for b in range(B):
            for d in range(1, N_DEV):
                rs_rdmas[3 * b + d - 1].wait_recv()
            acc_b = (partial_ref[b, my_pos].astype(jnp.float32)
                     + rs_comm[0, b].astype(jnp.float32)
                     + rs_comm[1, b].astype(jnp.float32)
                     + rs_comm[2, b].astype(jnp.float32))
            ag_send[b] = acc_b.astype(jnp.bfloat16)

            def _broadcast(rot, b=b):
                def _():
                    out_ref[b, :, rot * DBLK:(rot + 1) * DBLK] = ag_send[b]
                    for d in range(1, N_DEV):
                        pltpu.make_async_remote_copy(
                            src_ref=ag_send.at[b],
                            dst_ref=out_ref.at[b, :,
                                               pl.ds(rot * DBLK, DBLK)],
                            send_sem=ag_send_sems.at[d - 1, b],
                            recv_sem=ag_recv_sems.at[d - 1, b],
                            device_id=((my_pos + d) % N_DEV,),
                            device_id_type=pl.DeviceIdType.MESH,
                        ).start()
                return _
            for rot in range(N_DEV):
                pl.when(my_pos == rot)(_broadcast(rot))

            for d in range(1, N_DEV):
                ag_waits.append(pltpu.make_async_remote_copy(
                    src_ref=ag_send.at[b],
                    dst_ref=out_ref.at[b, :, pl.ds(0, DBLK)],
                    send_sem=ag_send_sems.at[d - 1, b],
                    recv_sem=ag_recv_sems.at[d - 1, b],
                    device_id=((my_pos + d) % N_DEV,),
                    device_id_type=pl.DeviceIdType.MESH,
                ))
        for rdma in ag_waits:
            rdma.wait_recv()
        for rdma in rs_rdmas:
            rdma.wait_send()
        for rdma in ag_waits:
            rdma.wait_send()

    return pl.pallas_call(
        body,
        grid=(1,),
        out_shape=jax.ShapeDtypeStruct((B, SQ, D_MODEL), jnp.bfloat16),
        in_specs=[
            pl.BlockSpec(memory_space=pltpu.VMEM),
            pl.BlockSpec(memory_space=pltpu.VMEM),
            pl.BlockSpec((B, SKV, H_LOC * DH),
                         lambda i: (0, 0, lax.axis_index("i"))),
            pl.BlockSpec((B, SKV, H_LOC * DH),
                         lambda i: (0, 0, lax.axis_index("i"))),
            pl.BlockSpec(memory_space=pltpu.VMEM),
        ],
        out_specs=pl.BlockSpec(memory_space=pltpu.VMEM),
        scratch_shapes=[
            pltpu.VMEM((B, N_DEV, SQ, DBLK), jnp.bfloat16),
            pltpu.VMEM((N_DEV - 1, B, SQ, DBLK), jnp.bfloat16),
            pltpu.VMEM((B, SQ, DBLK), jnp.bfloat16),
            pltpu.SemaphoreType.DMA((N_DEV - 1, B)),
            pltpu.SemaphoreType.DMA((N_DEV - 1, B)),
            pltpu.SemaphoreType.DMA((N_DEV - 1, B)),
            pltpu.SemaphoreType.DMA((N_DEV - 1, B)),
        ],
        compiler_params=pltpu.CompilerParams(collective_id=0),
    )(xb, Wq, k_flat, v_flat, Wo)


# device time: 16685 ns/iter; 1.1276x vs baseline; 1.1276x over previous
import jax
import jax.numpy as jnp
from jax import lax
from jax.experimental import pallas as pl
from jax.experimental.pallas import tpu as pltpu

N_DEV = 4
B, SQ, SKV, HQ, DH = 2, 256, 256, 16, 64
D_MODEL = 512
DBLK = D_MODEL // N_DEV
H_LOC = HQ // N_DEV
NB = SQ // 64


def kernel(x, Wq, K_ext, V_ext, Wo):
    my = lax.axis_index("i")
    xb = x.astype(jnp.bfloat16)
    k_loc = lax.dynamic_slice_in_dim(K_ext, my * H_LOC, H_LOC, axis=2
                                     ).astype(jnp.bfloat16)
    v_loc = lax.dynamic_slice_in_dim(V_ext, my * H_LOC, H_LOC, axis=2
                                     ).astype(jnp.bfloat16)

    def body(x_ref, wq_ref, k_ref, v_ref, wo_ref, out_ref,
             partial_ref, rs_comm, ag_send,
             rs_send_sems, rs_recv_sems, ag_send_sems, ag_recv_sems):
        my_pos = lax.axis_index("i")

        barrier = pltpu.get_barrier_semaphore()
        for d in range(1, N_DEV):
            pl.semaphore_signal(
                barrier, inc=1,
                device_id=((my_pos + d) % N_DEV,),
                device_id_type=pl.DeviceIdType.MESH,
            )

        wq = wq_ref[...].astype(jnp.bfloat16)
        wo = wo_ref[...].astype(jnp.bfloat16)

        rs_rdmas = []
        for b in range(B):
            q = jnp.dot(x_ref[b], wq,
                        preferred_element_type=jnp.float32)
            q4 = (q * 0.125).reshape(SQ, H_LOC, DH).astype(jnp.bfloat16)
            ctxs = []
            for h in range(H_LOC):
                q_blk = q4[:, h, :].reshape(NB, 64, DH)
                k_blk = k_ref[b, :, h, :].reshape(NB, 64, DH)
                s = lax.dot_general(
                    q_blk, k_blk, (((2,), (2,)), ((0,), (0,))),
                    preferred_element_type=jnp.float32,
                )
                w = jnp.exp(s)
                w = w / jnp.sum(w, axis=2, keepdims=True)
                v_blk = v_ref[b, :, h, :].reshape(NB, 64, DH)
                ctx = lax.dot_general(
                    w.astype(jnp.bfloat16), v_blk, (((2,), (1,)), ((0,), (0,))),
                    preferred_element_type=jnp.float32,
                )
                ctxs.append(ctx.reshape(SQ, DH).astype(jnp.bfloat16))
            ctx_b = jnp.concatenate(ctxs, axis=1)

            for t in range(N_DEV):
                p_t = jnp.dot(ctx_b, wo[:, t * DBLK:(t + 1) * DBLK],
                              preferred_element_type=jnp.float32)
                partial_ref[b, t] = p_t.astype(jnp.bfloat16)

            if b == 0:
                pl.semaphore_wait(barrier, N_DEV - 1)

            for d in range(1, N_DEV):
                rdma = pltpu.make_async_remote_copy(
                    src_ref=partial_ref.at[b, (my_pos + d) % N_DEV],
                    dst_ref=rs_comm.at[d - 1, b],
                    send_sem=rs_send_sems.at[d - 1, b],
                    recv_sem=rs_recv_sems.at[d - 1, b],
                    device_id=((my_pos + d) % N_DEV,),
                    device_id_type=pl.DeviceIdType.MESH,
                )
                rdma.start()
                rs_rdmas.append(rdma)

        ag_waits = []
        for b in range(B):
            for d in range(1, N_DEV):
                rs_rdmas[3 * b + d - 1].wait_recv()
            acc_b = (partial_ref[b, my_pos].astype(jnp.float32)
                     + rs_comm[0, b].astype(jnp.float32)
                     + rs_comm[1, b].astype(jnp.float32)
                     + rs_comm[2, b].astype(jnp.float32))
            ag_send[b] = acc_b.astype(jnp.bfloat16)

            def _broadcast(rot, b=b):
                def _():
                    out_ref[b, :, rot * DBLK:(rot + 1) * DBLK] = ag_send[b]
                    for d in range(1, N_DEV):
                        pltpu.make_async_remote_copy(
                            src_ref=ag_send.at[b],
                            dst_ref=out_ref.at[b, :,
                                               pl.ds(rot * DBLK, DBLK)],
                            send_sem=ag_send_sems.at[d - 1, b],
                            recv_sem=ag_recv_sems.at[d - 1, b],
                            device_id=((my_pos + d) % N_DEV,),
                            device_id_type=pl.DeviceIdType.MESH,
                        ).start()
                return _
            for rot in range(N_DEV):
                pl.when(my_pos == rot)(_broadcast(rot))

            for d in range(1, N_DEV):
                ag_waits.append(pltpu.make_async_remote_copy(
                    src_ref=ag_send.at[b],
                    dst_ref=out_ref.at[b, :, pl.ds(0, DBLK)],
                    send_sem=ag_send_sems.at[d - 1, b],
                    recv_sem=ag_recv_sems.at[d - 1, b],
                    device_id=((my_pos + d) % N_DEV,),
                    device_id_type=pl.DeviceIdType.MESH,
                ))
        for rdma in ag_waits:
            rdma.wait_recv()
        for rdma in rs_rdmas:
            rdma.wait_send()
        for rdma in ag_waits:
            rdma.wait_send()

    return pl.pallas_call(
        body,
        out_shape=jax.ShapeDtypeStruct((B, SQ, D_MODEL), jnp.bfloat16),
        in_specs=[pl.BlockSpec(memory_space=pltpu.VMEM)] * 5,
        out_specs=pl.BlockSpec(memory_space=pltpu.VMEM),
        scratch_shapes=[
            pltpu.VMEM((B, N_DEV, SQ, DBLK), jnp.bfloat16),
            pltpu.VMEM((N_DEV - 1, B, SQ, DBLK), jnp.bfloat16),
            pltpu.VMEM((B, SQ, DBLK), jnp.bfloat16),
            pltpu.SemaphoreType.DMA((N_DEV - 1, B)),
            pltpu.SemaphoreType.DMA((N_DEV - 1, B)),
            pltpu.SemaphoreType.DMA((N_DEV - 1, B)),
            pltpu.SemaphoreType.DMA((N_DEV - 1, B)),
        ],
        compiler_params=pltpu.CompilerParams(collective_id=0),
    )(xb, Wq, k_loc, v_loc, Wo)
